# baseline (speedup 1.0000x reference)
import functools

import jax
import jax.numpy as jnp
from jax import lax
from jax.experimental import pallas as pl
from jax.experimental.pallas import tpu as pltpu
from jax.experimental.pallas import tpu_sc as plsc

N = 100000
D = 128
G = 64
L = 16
NW = 32
RPW = 3120
CHUNK = 240
TAIL = N - NW * RPW
NK = D // L

_mesh = plsc.VectorSubcoreMesh(core_axis_name="c", subcore_axis_name="s")


@functools.partial(
    pl.kernel,
    out_type=[
        jax.ShapeDtypeStruct((NW, G, D), jnp.float32),
        jax.ShapeDtypeStruct((NW, G, L), jnp.float32),
    ],
    mesh=_mesh,
    compiler_params=pltpu.CompilerParams(needs_layout_passes=False),
    scratch_types=[
        pltpu.VMEM((CHUNK, D), jnp.float32),
        pltpu.VMEM((CHUNK,), jnp.int32),
        pltpu.VMEM((D,), jnp.float32),
        pltpu.VMEM((G, D), jnp.float32),
        pltpu.VMEM((G, L), jnp.float32),
    ],
)
def _sc_partials(x_hbm, w_hbm, batch_hbm, vout, sout, xb, bb, wv, vacc, sacc):
    wid = lax.axis_index("c") * 16 + lax.axis_index("s")

    pltpu.sync_copy(w_hbm.at[0], wv)
    wregs = [wv[pl.ds(k * L, L)] for k in range(NK)]
    zero = jnp.zeros((L,), jnp.float32)

    def _zero_seg(seg, _):
        for k in range(NK):
            vacc[seg, pl.ds(k * L, L)] = zero
        sacc[seg, :] = zero
        return 0

    lax.fori_loop(0, G, _zero_seg, 0)

    def _do_rows(row0, nrows):
        pltpu.sync_copy(x_hbm.at[pl.ds(row0, nrows), :], xb.at[pl.ds(0, nrows), :])
        pltpu.sync_copy(batch_hbm.at[pl.ds(row0, nrows)], bb.at[pl.ds(0, nrows)])

        def _group(g16, _):
            base = g16 * L
            sv = bb[pl.ds(base, L)]
            for i in range(L):
                row = base + i
                rvs = [xb[row, pl.ds(k * L, L)] for k in range(NK)]
                dp = rvs[0] * wregs[0]
                for k in range(1, NK):
                    dp = dp + rvs[k] * wregs[k]
                eb = jnp.exp(jnp.full((L,), jnp.sum(dp), jnp.float32))
                seg = sv[i]
                for k in range(NK):
                    plsc.addupdate(vacc.at[seg, pl.ds(k * L, L)], eb * rvs[k])
                plsc.addupdate(sacc.at[seg], eb)
            return 0

        lax.fori_loop(0, nrows // L, _group, 0)

    base = wid * RPW

    def _chunk(c, _):
        _do_rows(pl.multiple_of(base + c * CHUNK, L), CHUNK)
        return 0

    lax.fori_loop(0, RPW // CHUNK, _chunk, 0)

    @pl.when(wid == NW - 1)
    def _tail():
        _do_rows(NW * RPW, TAIL)

    pltpu.sync_copy(vacc, vout.at[wid])
    pltpu.sync_copy(sacc, sout.at[wid])


def _combine_body(v_ref, s_ref, o_ref):
    v = jnp.sum(v_ref[...], axis=0)
    s = jnp.sum(s_ref[...], axis=0)[:, 0:1]
    o_ref[...] = jnp.where(s > 0, v / s, 0.0)


def kernel(x, gate_w, gate_b, batch):
    del gate_b
    vp, sp = _sc_partials(x, gate_w, batch.astype(jnp.int32))
    out = pl.pallas_call(
        _combine_body,
        out_shape=jax.ShapeDtypeStruct((G, D), jnp.float32),
    )(vp, sp)
    return out
